# baseline (speedup 1.0000x reference)
import functools

import jax
import jax.numpy as jnp
from jax import lax
from jax.experimental import pallas as pl
from jax.experimental.pallas import tpu as pltpu
from jax.experimental.pallas import tpu_sc as plsc

_REG_WEIGHT = 1e-4

_NC = 2
_NS = 16
_NW = _NC * _NS
_CHUNK = 512


_BLK_T = 24576


def _transpose_body(x1_ref, x2_ref, o_ref):
    x = jnp.concatenate([x1_ref[...], x2_ref[...]], axis=0)
    o_ref[...] = jax.lax.dot_general(
        x, _eye(x.shape[0]), _DN0, preferred_element_type=jnp.float32)


def _tc_pack_transpose(table_t, half):
    d, n = table_t.shape
    grid = half // _BLK_T
    return pl.pallas_call(
        _transpose_body,
        grid=(grid,),
        in_specs=[
            pl.BlockSpec((d, _BLK_T), lambda i: (0, i)),
            pl.BlockSpec(
                (d, _BLK_T),
                lambda i, g=grid, m=(n - 1) // _BLK_T: (0, jnp.minimum(g + i, m)),
            ),
        ],
        out_specs=pl.BlockSpec((_BLK_T, 2 * d), lambda i: (i, 0)),
        out_shape=jax.ShapeDtypeStruct((half, 2 * d), jnp.float32),
        compiler_params=pltpu.CompilerParams(
            dimension_semantics=("parallel",),
        ),
    )(table_t, table_t)


def _sc_gather(table2, row_idx):
    B = row_idx.shape[0]
    D = table2.shape[1]
    chunks_per_w = B // (_NW * _CHUNK)
    mesh = plsc.VectorSubcoreMesh(core_axis_name="c", subcore_axis_name="s")

    @functools.partial(
        pl.kernel,
        mesh=mesh,
        out_type=jax.ShapeDtypeStruct((B, D), jnp.float32),
        scratch_types=[
            pltpu.VMEM((_CHUNK,), jnp.int32),
            pltpu.VMEM((_CHUNK, D), jnp.float32),
            pltpu.SemaphoreType.DMA,
        ],
    )
    def gather_kernel(table_hbm, idx_hbm, out_hbm, idx_v, rows_v, sem):
        wid = lax.axis_index("s") * _NC + lax.axis_index("c")

        @pl.loop(0, chunks_per_w)
        def _(c):
            base = (wid * chunks_per_w + c) * _CHUNK
            pltpu.sync_copy(idx_hbm.at[pl.ds(base, _CHUNK)], idx_v)
            pltpu.async_copy(table_hbm.at[idx_v], rows_v, sem).wait()
            pltpu.sync_copy(rows_v, out_hbm.at[pl.ds(base, _CHUNK)])

    return gather_kernel(table2, row_idx)


def _eye(d):
    return (jax.lax.broadcasted_iota(jnp.int32, (d, d), 0)
            == jax.lax.broadcasted_iota(jnp.int32, (d, d), 1)
            ).astype(jnp.float32)


_DN0 = (((0,), (0,)), ((), ()))


def _loss_body(inv_batch, u_ref, s_ref, gp_ref, gn_ref, o_ref):
    i = pl.program_id(0)
    ut = u_ref[...]
    s2 = s_ref[...]
    gp = gp_ref[...]
    gn = gn_ref[...]
    d = gp.shape[1] // 2
    u = jax.lax.dot_general(ut, _eye(d), _DN0,
                            preferred_element_type=jnp.float32)
    s = jax.lax.dot_general(s2, _eye(2), _DN0,
                            preferred_element_type=jnp.float32)
    sp = s[:, 0:1]
    sn = s[:, 1:2]
    p = gp[:, :d] + (gp[:, d:] - gp[:, :d]) * sp
    n = gn[:, :d] + (gn[:, d:] - gn[:, :d]) * sn
    diff = jnp.sum(u * (p - n), axis=1)
    loss_terms = -jnp.log(jax.nn.sigmoid(diff))
    l2 = jnp.sum(u * u + p * p + n * n, axis=1)
    part = jnp.sum(loss_terms + _REG_WEIGHT * l2) * inv_batch

    @pl.when(i == 0)
    def _():
        o_ref[...] = jnp.zeros((1, 1), jnp.float32)

    o_ref[...] += jnp.reshape(part, (1, 1))


def kernel(user_embeddings, item_embeddings, pos_items, neg_items):
    batch, d = user_embeddings.shape
    idx = jnp.concatenate(
        [pos_items.astype(jnp.int32), neg_items.astype(jnp.int32)]
    )
    num_items = item_embeddings.shape[0]
    half = (pl.cdiv(num_items, 2 * _BLK_T)) * _BLK_T
    in_hi = (idx >= half).astype(jnp.int32)
    row_idx = idx - half * in_hi
    table2 = _tc_pack_transpose(item_embeddings.T, half)
    rows = _sc_gather(table2, row_idx)

    sel2 = in_hi.astype(jnp.float32).reshape(2, batch)

    blk = 8192
    grid = batch // blk
    out = pl.pallas_call(
        functools.partial(_loss_body, 1.0 / batch),
        grid=(grid,),
        in_specs=[
            pl.BlockSpec((d, blk), lambda i: (0, i)),
            pl.BlockSpec((2, blk), lambda i: (0, i)),
            pl.BlockSpec((blk, 2 * d), lambda i: (i, 0)),
            pl.BlockSpec((blk, 2 * d), lambda i, g=grid: (i + g, 0)),
        ],
        out_specs=pl.BlockSpec((1, 1), lambda i: (0, 0)),
        out_shape=jax.ShapeDtypeStruct((1, 1), jnp.float32),
    )(user_embeddings.T, sel2, rows, rows)
    return out[0, 0]


# speedup vs baseline: 1.0065x; 1.0065x over previous
import functools

import jax
import jax.numpy as jnp
from jax import lax
from jax.experimental import pallas as pl
from jax.experimental.pallas import tpu as pltpu
from jax.experimental.pallas import tpu_sc as plsc

_REG_WEIGHT = 1e-4

_NC = 2
_NS = 16
_NW = _NC * _NS
_CHUNK = 512


_BLK_T = 16384


def _transpose_body(x1_ref, x2_ref, o_ref):
    x = jnp.concatenate([x1_ref[...], x2_ref[...]], axis=0)
    o_ref[...] = jax.lax.dot_general(
        x, _eye(x.shape[0]), _DN0, preferred_element_type=jnp.float32)


def _tc_pack_transpose(table_t, half):
    d, n = table_t.shape
    grid = half // _BLK_T
    return pl.pallas_call(
        _transpose_body,
        grid=(grid,),
        in_specs=[
            pl.BlockSpec((d, _BLK_T), lambda i: (0, i)),
            pl.BlockSpec(
                (d, _BLK_T),
                lambda i, g=grid, m=(n - 1) // _BLK_T: (0, jnp.minimum(g + i, m)),
            ),
        ],
        out_specs=pl.BlockSpec((_BLK_T, 2 * d), lambda i: (i, 0)),
        out_shape=jax.ShapeDtypeStruct((half, 2 * d), jnp.float32),
        compiler_params=pltpu.CompilerParams(
            dimension_semantics=("parallel",),
        ),
    )(table_t, table_t)


def _sc_gather(table2, row_idx):
    B = row_idx.shape[0]
    D = table2.shape[1]
    chunks_per_w = B // (_NW * _CHUNK)
    mesh = plsc.VectorSubcoreMesh(core_axis_name="c", subcore_axis_name="s")

    @functools.partial(
        pl.kernel,
        mesh=mesh,
        out_type=jax.ShapeDtypeStruct((B, D), jnp.float32),
        scratch_types=[
            pltpu.VMEM((_CHUNK,), jnp.int32),
            pltpu.VMEM((_CHUNK, D), jnp.float32),
            pltpu.SemaphoreType.DMA,
        ],
    )
    def gather_kernel(table_hbm, idx_hbm, out_hbm, idx_v, rows_v, sem):
        wid = lax.axis_index("s") * _NC + lax.axis_index("c")

        @pl.loop(0, chunks_per_w)
        def _(c):
            base = (wid * chunks_per_w + c) * _CHUNK
            pltpu.sync_copy(idx_hbm.at[pl.ds(base, _CHUNK)], idx_v)
            pltpu.async_copy(table_hbm.at[idx_v], rows_v, sem).wait()
            pltpu.sync_copy(rows_v, out_hbm.at[pl.ds(base, _CHUNK)])

    return gather_kernel(table2, row_idx)


def _eye(d):
    return (jax.lax.broadcasted_iota(jnp.int32, (d, d), 0)
            == jax.lax.broadcasted_iota(jnp.int32, (d, d), 1)
            ).astype(jnp.float32)


_DN0 = (((0,), (0,)), ((), ()))


def _loss_body(inv_batch, u_ref, s_ref, gp_ref, gn_ref, o_ref):
    i = pl.program_id(0)
    ut = u_ref[...]
    s2 = s_ref[...]
    gp = gp_ref[...]
    gn = gn_ref[...]
    d = gp.shape[1] // 2
    u = jax.lax.dot_general(ut, _eye(d), _DN0,
                            preferred_element_type=jnp.float32)
    s = jax.lax.dot_general(s2, _eye(2), _DN0,
                            preferred_element_type=jnp.float32)
    sp = s[:, 0:1]
    sn = s[:, 1:2]
    p = gp[:, :d] + (gp[:, d:] - gp[:, :d]) * sp
    n = gn[:, :d] + (gn[:, d:] - gn[:, :d]) * sn
    diff = jnp.sum(u * (p - n), axis=1)
    loss_terms = -jnp.log(jax.nn.sigmoid(diff))
    l2 = jnp.sum(u * u + p * p + n * n, axis=1)
    part = jnp.sum(loss_terms + _REG_WEIGHT * l2) * inv_batch

    @pl.when(i == 0)
    def _():
        o_ref[...] = jnp.zeros((1, 1), jnp.float32)

    o_ref[...] += jnp.reshape(part, (1, 1))


def kernel(user_embeddings, item_embeddings, pos_items, neg_items):
    batch, d = user_embeddings.shape
    idx = jnp.concatenate(
        [pos_items.astype(jnp.int32), neg_items.astype(jnp.int32)]
    )
    num_items = item_embeddings.shape[0]
    half = (pl.cdiv(num_items, 2 * _BLK_T)) * _BLK_T
    in_hi = (idx >= half).astype(jnp.int32)
    row_idx = idx - half * in_hi
    table2 = _tc_pack_transpose(item_embeddings.T, half)
    rows = _sc_gather(table2, row_idx)

    sel2 = in_hi.astype(jnp.float32).reshape(2, batch)

    blk = 4096
    grid = batch // blk
    out = pl.pallas_call(
        functools.partial(_loss_body, 1.0 / batch),
        grid=(grid,),
        in_specs=[
            pl.BlockSpec((d, blk), lambda i: (0, i)),
            pl.BlockSpec((2, blk), lambda i: (0, i)),
            pl.BlockSpec((blk, 2 * d), lambda i: (i, 0)),
            pl.BlockSpec((blk, 2 * d), lambda i, g=grid: (i + g, 0)),
        ],
        out_specs=pl.BlockSpec((1, 1), lambda i: (0, 0)),
        out_shape=jax.ShapeDtypeStruct((1, 1), jnp.float32),
    )(user_embeddings.T, sel2, rows, rows)
    return out[0, 0]
